# baseline (speedup 1.0000x reference)
import functools

import jax
import jax.numpy as jnp
from jax import lax
from jax.experimental import pallas as pl
from jax.experimental.pallas import tpu as pltpu
from jax.experimental.pallas import tpu_sc as plsc

LANES = 16
GROUP = 16


def _make_kernel(B, S, D, Dp, Sp):
    NC, NS = 2, 16
    NW = NC * NS
    assert B % NW == 0
    nseg = B // NW
    assert nseg % GROUP == 0 and nseg % 2 == 0
    n_full = D // LANES
    offs = tuple(range(0, n_full * LANES, LANES))
    if D % LANES:
        offs = offs + (D - LANES,)
    inv_s = jnp.float32(1.0 / S)

    mesh = plsc.VectorSubcoreMesh(core_axis_name="c", subcore_axis_name="s")
    out_sds = jax.ShapeDtypeStruct((B, D), jnp.float32)

    @functools.partial(
        pl.kernel,
        out_type=(out_sds, out_sds),
        mesh=mesh,
        scratch_types=[
            pltpu.VMEM((nseg, Sp), jnp.int32),
            pltpu.VMEM((Sp, Dp), jnp.float32),
            pltpu.VMEM((Sp, Dp), jnp.float32),
            pltpu.VMEM((GROUP, D), jnp.float32),
            pltpu.SemaphoreType.DMA,
            pltpu.SemaphoreType.DMA,
        ],
        compiler_params=pltpu.CompilerParams(use_tc_tiling_on_sc=True,
                                             needs_layout_passes=False),
    )
    def k(prem_hbm, hyp_hbm, table_hbm, out_p, out_h,
          idx_v, rows0, rows1, stage, sem0, sem1):
        wid = lax.axis_index("s") * NC + lax.axis_index("c")
        base = wid * nseg
        bufs = (rows0, rows1)
        sems = (sem0, sem1)

        def start(g, buf, sem):
            pltpu.make_async_copy(table_hbm.at[idx_v.at[g]], buf, sem).start()

        def wait(g, buf, sem):
            pltpu.make_async_copy(table_hbm.at[idx_v.at[g]], buf, sem).wait()

        def process(idx_hbm, out_hbm):
            pltpu.sync_copy(idx_hbm.at[pl.ds(pl.multiple_of(base, 8), nseg)],
                            idx_v)
            start(0, bufs[0], sems[0])
            start(1, bufs[1], sems[1])

            def outer(g2, carry):
                for b in range(2):
                    gg = g2 * 2 + b
                    wait(gg, bufs[b], sems[b])
                    buf = bufs[b]

                    def srow(s, accs):
                        return tuple(
                            a + buf[s, pl.ds(o, LANES)]
                            for a, o in zip(accs, offs)
                        )

                    accs = lax.fori_loop(
                        0, S, srow,
                        tuple(jnp.zeros((LANES,), jnp.float32) for _ in offs),
                    )

                    @pl.when(gg + 2 < nseg)
                    def _():
                        start(gg + 2, bufs[b], sems[b])

                    row = lax.rem(gg, GROUP)
                    for a, o in zip(accs, offs):
                        stage[row, pl.ds(o, LANES)] = a * inv_s

                    @pl.when(row == GROUP - 1)
                    def _():
                        flush_base = pl.multiple_of(
                            base + gg - (GROUP - 1), 8)
                        pltpu.sync_copy(
                            stage, out_hbm.at[pl.ds(flush_base, GROUP)])
                return carry

            lax.fori_loop(0, nseg // 2, outer, 0)

        process(prem_hbm, out_p)
        process(hyp_hbm, out_h)

    return k


def kernel(premises, hypothesis, glove_embeddings):
    B, S = premises.shape
    V, D = glove_embeddings.shape
    Dp = (D + 127) // 128 * 128
    Sp = (S + 7) // 8 * 8
    if Dp != D:
        glove_embeddings = jnp.pad(glove_embeddings, ((0, 0), (0, Dp - D)))
    if Sp != S:
        premises = jnp.pad(premises, ((0, 0), (0, Sp - S)))
        hypothesis = jnp.pad(hypothesis, ((0, 0), (0, Sp - S)))
    k = _make_kernel(B, S, D, Dp, Sp)
    return k(premises, hypothesis, glove_embeddings)


# speedup vs baseline: 1.7091x; 1.7091x over previous
import functools

import jax
import jax.numpy as jnp
from jax import lax
from jax.experimental import pallas as pl
from jax.experimental.pallas import tpu as pltpu
from jax.experimental.pallas import tpu_sc as plsc

LANES = 16
GROUP = 16


def _make_kernel(B, S, D):
    NC, NS = 2, 16
    NW = NC * NS
    assert B % NW == 0
    nseg = B // NW
    assert nseg % GROUP == 0 and nseg % 2 == 0
    n_full = D // LANES
    offs = tuple(range(0, n_full * LANES, LANES))
    if D % LANES:
        offs = offs + (D - LANES,)
    inv_s = jnp.float32(1.0 / S)

    mesh = plsc.VectorSubcoreMesh(core_axis_name="c", subcore_axis_name="s")
    out_sds = jax.ShapeDtypeStruct((B, D), jnp.float32)

    @functools.partial(
        pl.kernel,
        out_type=(out_sds, out_sds),
        mesh=mesh,
        scratch_types=[
            pltpu.VMEM((nseg, S), jnp.int32),
            pltpu.VMEM((S, D), jnp.float32),
            pltpu.VMEM((S, D), jnp.float32),
            pltpu.VMEM((GROUP, D), jnp.float32),
            pltpu.SemaphoreType.DMA,
            pltpu.SemaphoreType.DMA,
        ],
        compiler_params=pltpu.CompilerParams(use_tc_tiling_on_sc=False,
                                             needs_layout_passes=False),
    )
    def k(prem_hbm, hyp_hbm, table_hbm, out_p, out_h,
          idx_v, rows0, rows1, stage, sem0, sem1):
        wid = lax.axis_index("s") * NC + lax.axis_index("c")
        base = wid * nseg
        bufs = (rows0, rows1)
        sems = (sem0, sem1)
        chunk_starts = tuple(range(0, S - LANES + 1, LANES))
        if (S - LANES) % LANES:
            chunk_starts = chunk_starts + (S - LANES,)

        def start(g, b):
            chunks = {o: idx_v[g, pl.ds(o, LANES)] for o in chunk_starts}
            for j in range(S):
                o = (j // LANES) * LANES
                if o not in chunks:
                    o = S - LANES
                v = chunks[o][j - o]
                pltpu.make_async_copy(
                    table_hbm.at[pl.ds(v, 1)],
                    bufs[b].at[pl.ds(j, 1)],
                    sems[b],
                ).start()

        def wait(b):
            pltpu.make_async_copy(
                table_hbm.at[pl.ds(0, S)], bufs[b], sems[b]).wait()

        def process(idx_hbm, out_hbm):
            pltpu.sync_copy(idx_hbm.at[pl.ds(pl.multiple_of(base, 8), nseg)],
                            idx_v)
            start(0, 0)
            start(1, 1)

            def outer(g2, carry):
                for b in range(2):
                    gg = g2 * 2 + b
                    wait(b)
                    buf = bufs[b]

                    def srow(s, accs):
                        return tuple(
                            a + buf[s, pl.ds(o, LANES)]
                            for a, o in zip(accs, offs)
                        )

                    accs = lax.fori_loop(
                        0, S, srow,
                        tuple(jnp.zeros((LANES,), jnp.float32) for _ in offs),
                    )

                    @pl.when(gg + 2 < nseg)
                    def _():
                        start(gg + 2, b)

                    row = lax.rem(gg, GROUP)
                    for a, o in zip(accs, offs):
                        stage[row, pl.ds(o, LANES)] = a * inv_s

                    @pl.when(row == GROUP - 1)
                    def _():
                        flush_base = pl.multiple_of(
                            base + gg - (GROUP - 1), 8)
                        pltpu.sync_copy(
                            stage, out_hbm.at[pl.ds(flush_base, GROUP)])
                return carry

            lax.fori_loop(0, nseg // 2, outer, 0)

        process(prem_hbm, out_p)
        process(hyp_hbm, out_h)

    return k


def kernel(premises, hypothesis, glove_embeddings):
    B, S = premises.shape
    V, D = glove_embeddings.shape
    k = _make_kernel(B, S, D)
    return k(premises, hypothesis, glove_embeddings)


# speedup vs baseline: 1.7119x; 1.0016x over previous
import functools

import jax
import jax.numpy as jnp
from jax import lax
from jax.experimental import pallas as pl
from jax.experimental.pallas import tpu as pltpu
from jax.experimental.pallas import tpu_sc as plsc

LANES = 16
GROUP = 16


def _make_kernel(B, S, D, Dp):
    NC, NS = 2, 16
    NW = NC * NS
    assert B % NW == 0
    nseg = B // NW
    assert nseg % GROUP == 0 and nseg % 2 == 0
    n_full = D // LANES
    offs = tuple(range(0, n_full * LANES, LANES))
    if D % LANES:
        offs = offs + (D - LANES,)
    inv_s = jnp.float32(1.0 / S)

    mesh = plsc.VectorSubcoreMesh(core_axis_name="c", subcore_axis_name="s")
    out_sds = jax.ShapeDtypeStruct((B, D), jnp.float32)

    @functools.partial(
        pl.kernel,
        out_type=(out_sds, out_sds),
        mesh=mesh,
        scratch_types=[
            pltpu.VMEM((nseg, S), jnp.int32),
            pltpu.VMEM((S, Dp), jnp.float32),
            pltpu.VMEM((S, Dp), jnp.float32),
            pltpu.VMEM((GROUP, D), jnp.float32),
            pltpu.SemaphoreType.DMA,
            pltpu.SemaphoreType.DMA,
        ],
        compiler_params=pltpu.CompilerParams(use_tc_tiling_on_sc=False,
                                            needs_layout_passes=False),
    )
    def k(prem_hbm, hyp_hbm, table_hbm, out_p, out_h,
          idx_v, rows0, rows1, stage, sem0, sem1):
        wid = lax.axis_index("s") * NC + lax.axis_index("c")
        base = wid * nseg
        bufs = (rows0, rows1)
        sems = (sem0, sem1)

        def start(g, buf, sem):
            pltpu.make_async_copy(table_hbm.at[idx_v.at[g]], buf, sem).start()

        def wait(g, buf, sem):
            pltpu.make_async_copy(table_hbm.at[idx_v.at[g]], buf, sem).wait()

        def process(idx_hbm, out_hbm):
            pltpu.sync_copy(idx_hbm.at[pl.ds(pl.multiple_of(base, 8), nseg)],
                            idx_v)
            start(0, bufs[0], sems[0])
            start(1, bufs[1], sems[1])

            def outer(g2, carry):
                for b in range(2):
                    gg = g2 * 2 + b
                    wait(gg, bufs[b], sems[b])
                    buf = bufs[b]

                    def srow(s, accs):
                        return tuple(
                            a + buf[s, pl.ds(o, LANES)]
                            for a, o in zip(accs, offs)
                        )

                    accs = lax.fori_loop(
                        0, S, srow,
                        tuple(jnp.zeros((LANES,), jnp.float32) for _ in offs),
                    )

                    @pl.when(gg + 2 < nseg)
                    def _():
                        start(gg + 2, bufs[b], sems[b])

                    row = lax.rem(gg, GROUP)
                    for a, o in zip(accs, offs):
                        stage[row, pl.ds(o, LANES)] = a * inv_s

                    @pl.when(row == GROUP - 1)
                    def _():
                        flush_base = pl.multiple_of(
                            base + gg - (GROUP - 1), 8)
                        pltpu.sync_copy(
                            stage, out_hbm.at[pl.ds(flush_base, GROUP)])
                return carry

            lax.fori_loop(0, nseg // 2, outer, 0)

        process(prem_hbm, out_p)
        process(hyp_hbm, out_h)

    return k


def kernel(premises, hypothesis, glove_embeddings):
    B, S = premises.shape
    V, D = glove_embeddings.shape
    Dp = (D + 7) // 8 * 8
    if Dp != D:
        glove_embeddings = jnp.pad(glove_embeddings, ((0, 0), (0, Dp - D)))
    k = _make_kernel(B, S, D, Dp)
    return k(premises, hypothesis, glove_embeddings)
